# baseline (speedup 1.0000x reference)
import functools

import jax
import jax.numpy as jnp
from jax import lax
from jax.experimental import pallas as pl
from jax.experimental.pallas import tpu as pltpu
from jax.experimental.pallas import tpu_sc as plsc

N = 32768
D_IN = 256
D_HID = 128
D_ATT = 64
B = 16
BLK = 1024

_NC = 2
_NS = 16
_TILES = _NC * _NS
_RPT = N // _TILES
_SUB = 256
_NSUB = _RPT // _SUB
_LANE = 16
_NV = D_HID // _LANE


def _enc_body(starts_ref, ends_ref, x_ref, W_enc_ref, b_enc_ref, V_ref,
              w_att_ref, g_ref, s_ref, s_acc):
    blk = pl.program_id(0)
    nblk = pl.num_programs(0)

    x = x_ref[...]
    h = jnp.dot(x, W_enc_ref[...], preferred_element_type=jnp.float32)
    h = h + b_enc_ref[...]
    t = jnp.tanh(jnp.dot(h, V_ref[...], preferred_element_type=jnp.float32))
    a = jnp.dot(t, w_att_ref[...], preferred_element_type=jnp.float32)
    e = jnp.exp(a)
    g_ref[...] = h * e

    i = blk * BLK + lax.broadcasted_iota(jnp.int32, (BLK, 1), 0)
    m = (i >= starts_ref[...]) & (i < ends_ref[...])
    me = jnp.where(m, e, 0.0)
    ones = jnp.ones((BLK, 1), jnp.float32)
    sp = lax.dot_general(me, ones, (((0,), (0,)), ((), ())),
                         preferred_element_type=jnp.float32)

    @pl.when(blk == 0)
    def _():
        s_acc[...] = sp

    @pl.when(blk > 0)
    def _():
        s_acc[...] += sp

    @pl.when(blk == nblk - 1)
    def _():
        s_ref[...] = s_acc[...]


def _encode(x, starts, ends, W_enc, b_enc, V, w_att):
    nblk = N // BLK
    return pl.pallas_call(
        _enc_body,
        grid=(nblk,),
        in_specs=[
            pl.BlockSpec((1, B), lambda i: (0, 0)),
            pl.BlockSpec((1, B), lambda i: (0, 0)),
            pl.BlockSpec((BLK, D_IN), lambda i: (i, 0)),
            pl.BlockSpec((D_IN, D_HID), lambda i: (0, 0)),
            pl.BlockSpec((1, D_HID), lambda i: (0, 0)),
            pl.BlockSpec((D_HID, D_ATT), lambda i: (0, 0)),
            pl.BlockSpec((D_ATT, 1), lambda i: (0, 0)),
        ],
        out_specs=[
            pl.BlockSpec((BLK, D_HID), lambda i: (i, 0)),
            pl.BlockSpec((B, 1), lambda i: (0, 0)),
        ],
        out_shape=[
            jax.ShapeDtypeStruct((N, D_HID), jnp.float32),
            jax.ShapeDtypeStruct((B, 1), jnp.float32),
        ],
        scratch_shapes=[pltpu.VMEM((B, 1), jnp.float32)],
    )(starts, ends, x, W_enc, b_enc.reshape(1, D_HID), V, w_att)


_SC_MESH = plsc.VectorSubcoreMesh(core_axis_name="c", subcore_axis_name="s",
                                  num_cores=_NC, num_subcores=_NS)


@functools.partial(
    pl.kernel,
    out_type=jax.ShapeDtypeStruct((_TILES, B, D_HID), jnp.float32),
    mesh=_SC_MESH,
    scratch_types=[
        pltpu.VMEM((2, _SUB, D_HID), jnp.float32),
        pltpu.VMEM((B, D_HID), jnp.float32),
        pltpu.VMEM((32,), jnp.int32),
        pltpu.SemaphoreType.DMA,
        pltpu.SemaphoreType.DMA,
    ],
)
def _sc_segment_sum(g_hbm, bs_hbm, zp_hbm, hbuf, z_acc, bs_v, sem0, sem1):
    cid = lax.axis_index("c")
    sid = lax.axis_index("s")
    wid = sid * _NC + cid
    lo = wid * _RPT

    pltpu.sync_copy(bs_hbm, bs_v)
    bs_lo = bs_v[pl.ds(0, 16)]
    bs_hi = bs_v[pl.ds(16, 16)]

    sems = (sem0, sem1)
    descs = [None] * _NSUB
    descs[0] = pltpu.async_copy(g_hbm.at[pl.ds(lo, _SUB)], hbuf.at[0], sem0)

    zero = jnp.zeros((_LANE,), jnp.float32)
    for j in range(B):
        for c in range(_NV):
            z_acc[j, pl.ds(c * _LANE, _LANE)] = zero

    for sc in range(_NSUB):
        buf = sc % 2
        if sc + 1 < _NSUB:
            descs[sc + 1] = pltpu.async_copy(
                g_hbm.at[pl.ds(lo + (sc + 1) * _SUB, _SUB)],
                hbuf.at[(sc + 1) % 2], sems[(sc + 1) % 2])
        descs[sc].wait()

        sub_lo = lo + sc * _SUB
        sub_hi = sub_lo + _SUB
        for j in range(B):
            r0 = jnp.maximum(sub_lo, bs_lo[j])
            r1 = jnp.minimum(sub_hi, bs_hi[j])

            @pl.when(r1 > r0)
            def _(j=j, r0=r0, r1=r1, buf=buf, sub_lo=sub_lo):
                def body(r, acc):
                    row = r - sub_lo
                    return tuple(
                        acc[c] + hbuf[buf, row, pl.ds(c * _LANE, _LANE)]
                        for c in range(_NV))

                acc0 = tuple(jnp.zeros((_LANE,), jnp.float32)
                             for _ in range(_NV))
                acc = lax.fori_loop(r0, r1, body, acc0)
                for c in range(_NV):
                    sl = pl.ds(c * _LANE, _LANE)
                    z_acc[j, sl] = z_acc[j, sl] + acc[c]

    pltpu.sync_copy(z_acc, zp_hbm.at[wid])


def _fin_body(zp_ref, s_ref, W_bag_ref, b_bag_ref, out_ref):
    z = jnp.sum(zp_ref[...], axis=0)
    num = lax.dot_general(z, W_bag_ref[...], (((1,), (0,)), ((), ())),
                          preferred_element_type=jnp.float32)
    s = s_ref[...]
    out_ref[...] = num / jnp.where(s > 0, s, 1.0) + b_bag_ref[...]


def _finalize(z_part, s, W_bag, b_bag):
    return pl.pallas_call(
        _fin_body,
        out_shape=jax.ShapeDtypeStruct((B, 1), jnp.float32),
    )(z_part, s, W_bag, b_bag.reshape(1, 1))


def kernel(x, bag_sizes, W_enc, b_enc, V, w_att, W_ins, b_ins, W_bag, b_bag):
    starts = bag_sizes[:B].reshape(1, B)
    ends = bag_sizes[1:].reshape(1, B)
    bs_pad = jnp.concatenate([bag_sizes[:B], bag_sizes[1:]])
    g, s = _encode(x, starts, ends, W_enc, b_enc, V, w_att)
    z_part = _sc_segment_sum(g, bs_pad)
    return _finalize(z_part, s, W_bag, b_bag)


# speedup vs baseline: 1.3252x; 1.3252x over previous
import functools

import jax
import jax.numpy as jnp
from jax import lax
from jax.experimental import pallas as pl
from jax.experimental.pallas import tpu as pltpu
from jax.experimental.pallas import tpu_sc as plsc

N = 32768
D_IN = 256
D_HID = 128
D_ATT = 64
B = 16
BLK = 1024
NBLK = N // BLK

_NC = 2
_NS = 16
_TILES = _NC * _NS
_RPT = N // _TILES
_LANE = 16
_VPT = _RPT // _LANE


def _enc_body(x_ref, W_enc_ref, b_enc_ref, V_ref, w_att_ref, W_bag_ref,
              g_ref):
    x = x_ref[...]
    h = jnp.dot(x, W_enc_ref[...], preferred_element_type=jnp.float32)
    h = h + b_enc_ref[...]
    t = jnp.tanh(jnp.dot(h, V_ref[...], preferred_element_type=jnp.float32))
    a_row = lax.dot_general(w_att_ref[...], t, (((0,), (1,)), ((), ())),
                            preferred_element_type=jnp.float32)
    e_row = jnp.exp(a_row)
    p_row = lax.dot_general(W_bag_ref[...], h, (((0,), (1,)), ((), ())),
                            preferred_element_type=jnp.float32)
    g_ref[...] = jnp.concatenate([e_row * p_row, e_row], axis=0)


def _encode(x, W_enc, b_enc, V, w_att, W_bag):
    return pl.pallas_call(
        _enc_body,
        grid=(NBLK,),
        in_specs=[
            pl.BlockSpec((BLK, D_IN), lambda i: (i, 0)),
            pl.BlockSpec((D_IN, D_HID), lambda i: (0, 0)),
            pl.BlockSpec((1, D_HID), lambda i: (0, 0)),
            pl.BlockSpec((D_HID, D_ATT), lambda i: (0, 0)),
            pl.BlockSpec((D_ATT, 1), lambda i: (0, 0)),
            pl.BlockSpec((D_HID, 1), lambda i: (0, 0)),
        ],
        out_specs=pl.BlockSpec((2, BLK), lambda i: (0, i)),
        out_shape=jax.ShapeDtypeStruct((2, N), jnp.float32),
    )(x, W_enc, b_enc.reshape(1, D_HID), V, w_att, W_bag)


_SC_MESH = plsc.VectorSubcoreMesh(core_axis_name="c", subcore_axis_name="s",
                                  num_cores=_NC, num_subcores=_NS)


@functools.partial(
    pl.kernel,
    out_type=jax.ShapeDtypeStruct((_TILES, 2, B, _LANE), jnp.float32),
    mesh=_SC_MESH,
    scratch_types=[
        pltpu.VMEM((_RPT,), jnp.float32),
        pltpu.VMEM((_RPT,), jnp.float32),
        pltpu.VMEM((B, _LANE), jnp.float32),
        pltpu.VMEM((B, _LANE), jnp.float32),
        pltpu.VMEM((2 * B,), jnp.int32),
    ],
)
def _sc_bag_sums(g_hbm, bs_hbm, out_hbm, wbuf, ebuf, acc_w, acc_e, bs_v):
    cid = lax.axis_index("c")
    sid = lax.axis_index("s")
    wid = sid * _NC + cid
    lo = wid * _RPT

    pltpu.sync_copy(bs_hbm, bs_v)
    pltpu.sync_copy(g_hbm.at[0, pl.ds(lo, _RPT)], wbuf)
    pltpu.sync_copy(g_hbm.at[1, pl.ds(lo, _RPT)], ebuf)
    bs_lo = bs_v[pl.ds(0, B)]
    bs_hi = bs_v[pl.ds(B, B)]

    lane = lax.iota(jnp.int32, _LANE)
    zero = jnp.zeros((_LANE,), jnp.float32)
    for j in range(B):
        acc_w[j] = zero
        acc_e[j] = zero

    for j in range(B):
        r0 = jnp.maximum(lo, bs_lo[j])
        r1 = jnp.minimum(lo + _RPT, bs_hi[j])

        @pl.when(r1 > r0)
        def _(j=j, r0=r0, r1=r1):
            v0 = (r0 - lo) // _LANE
            v1 = (r1 - lo + _LANE - 1) // _LANE

            def body(v, accs):
                aw, ae = accs
                base = v * _LANE
                idx = lo + base + lane
                m = (idx >= r0) & (idx < r1)
                wv = wbuf[pl.ds(base, _LANE)]
                ev = ebuf[pl.ds(base, _LANE)]
                return (aw + jnp.where(m, wv, 0.0),
                        ae + jnp.where(m, ev, 0.0))

            aw, ae = lax.fori_loop(v0, v1, body, (zero, zero))
            acc_w[j] = aw
            acc_e[j] = ae

    pltpu.sync_copy(acc_w, out_hbm.at[wid, 0])
    pltpu.sync_copy(acc_e, out_hbm.at[wid, 1])


def _fin_body(zp_ref, b_bag_ref, out_ref):
    zp = zp_ref[...]
    num = jnp.sum(zp[:, 0, :, :], axis=(0, 2))
    den = jnp.sum(zp[:, 1, :, :], axis=(0, 2))
    yhat = num / jnp.where(den > 0, den, 1.0) + b_bag_ref[0, 0]
    out_ref[...] = yhat.reshape(B, 1)


def _finalize(zp, b_bag):
    return pl.pallas_call(
        _fin_body,
        out_shape=jax.ShapeDtypeStruct((B, 1), jnp.float32),
    )(zp, b_bag.reshape(1, 1))


def kernel(x, bag_sizes, W_enc, b_enc, V, w_att, W_ins, b_ins, W_bag, b_bag):
    bs_pad = jnp.concatenate([bag_sizes[:B], bag_sizes[1:]])
    g = _encode(x, W_enc, b_enc, V, w_att, W_bag)
    zp = _sc_bag_sums(g, bs_pad)
    return _finalize(zp, b_bag)


# speedup vs baseline: 1.5038x; 1.1348x over previous
import functools

import jax
import jax.numpy as jnp
from jax import lax
from jax.experimental import pallas as pl
from jax.experimental.pallas import tpu as pltpu
from jax.experimental.pallas import tpu_sc as plsc

N = 32768
D_IN = 256
D_HID = 128
D_ATT = 64
B = 16
BLK = 2048
NBLK = N // BLK

_NC = 2
_NS = 16
_TILES = _NC * _NS
_RPT = N // _TILES
_LANE = 16
_VPT = _RPT // _LANE


def _enc_body(x_ref, W_enc_ref, b_enc_ref, V_ref, w_att_ref, W_bag_ref,
              g_ref):
    x = x_ref[...].astype(jnp.bfloat16)
    h = jnp.dot(x, W_enc_ref[...], preferred_element_type=jnp.float32)
    h = h + b_enc_ref[...]
    t = jnp.tanh(jnp.dot(h.astype(jnp.bfloat16), V_ref[...],
                         preferred_element_type=jnp.float32))
    a_row = lax.dot_general(w_att_ref[...], t, (((0,), (1,)), ((), ())),
                            preferred_element_type=jnp.float32)
    e_row = jnp.exp(a_row)
    p_row = lax.dot_general(W_bag_ref[...], h, (((0,), (1,)), ((), ())),
                            preferred_element_type=jnp.float32)
    g_ref[...] = jnp.concatenate([e_row * p_row, e_row], axis=0)


def _encode(x, W_enc, b_enc, V, w_att, W_bag):
    return pl.pallas_call(
        _enc_body,
        grid=(NBLK,),
        in_specs=[
            pl.BlockSpec((BLK, D_IN), lambda i: (i, 0)),
            pl.BlockSpec((D_IN, D_HID), lambda i: (0, 0)),
            pl.BlockSpec((1, D_HID), lambda i: (0, 0)),
            pl.BlockSpec((D_HID, D_ATT), lambda i: (0, 0)),
            pl.BlockSpec((D_ATT, 1), lambda i: (0, 0)),
            pl.BlockSpec((D_HID, 1), lambda i: (0, 0)),
        ],
        out_specs=pl.BlockSpec((2, BLK), lambda i: (0, i)),
        out_shape=jax.ShapeDtypeStruct((2, N), jnp.float32),
    )(x, W_enc.astype(jnp.bfloat16), b_enc.reshape(1, D_HID),
      V.astype(jnp.bfloat16), w_att, W_bag)


_SC_MESH = plsc.VectorSubcoreMesh(core_axis_name="c", subcore_axis_name="s",
                                  num_cores=_NC, num_subcores=_NS)


@functools.partial(
    pl.kernel,
    out_type=jax.ShapeDtypeStruct((_TILES, 2, B, _LANE), jnp.float32),
    mesh=_SC_MESH,
    scratch_types=[
        pltpu.VMEM((_RPT,), jnp.float32),
        pltpu.VMEM((_RPT,), jnp.float32),
        pltpu.VMEM((B, _LANE), jnp.float32),
        pltpu.VMEM((B, _LANE), jnp.float32),
        pltpu.VMEM((2 * B,), jnp.int32),
    ],
)
def _sc_bag_sums(g_hbm, bs_hbm, out_hbm, wbuf, ebuf, acc_w, acc_e, bs_v):
    cid = lax.axis_index("c")
    sid = lax.axis_index("s")
    wid = sid * _NC + cid
    lo = wid * _RPT

    pltpu.sync_copy(bs_hbm, bs_v)
    pltpu.sync_copy(g_hbm.at[0, pl.ds(lo, _RPT)], wbuf)
    pltpu.sync_copy(g_hbm.at[1, pl.ds(lo, _RPT)], ebuf)
    bs_lo = bs_v[pl.ds(0, B)]
    bs_hi = bs_v[pl.ds(B, B)]

    lane = lax.iota(jnp.int32, _LANE)
    zero = jnp.zeros((_LANE,), jnp.float32)
    for j in range(B):
        acc_w[j] = zero
        acc_e[j] = zero

    for j in range(B):
        r0 = jnp.maximum(lo, bs_lo[j])
        r1 = jnp.minimum(lo + _RPT, bs_hi[j])

        @pl.when(r1 > r0)
        def _(j=j, r0=r0, r1=r1):
            v0 = (r0 - lo) // _LANE
            v1 = (r1 - lo + _LANE - 1) // _LANE

            def body(v, accs):
                aw, ae = accs
                base = v * _LANE
                idx = lo + base + lane
                m = (idx >= r0) & (idx < r1)
                wv = wbuf[pl.ds(base, _LANE)]
                ev = ebuf[pl.ds(base, _LANE)]
                return (aw + jnp.where(m, wv, 0.0),
                        ae + jnp.where(m, ev, 0.0))

            aw, ae = lax.fori_loop(v0, v1, body, (zero, zero))
            acc_w[j] = aw
            acc_e[j] = ae

    pltpu.sync_copy(acc_w, out_hbm.at[wid, 0])
    pltpu.sync_copy(acc_e, out_hbm.at[wid, 1])


def _fin_body(zp_ref, b_bag_ref, out_ref):
    zp = zp_ref[...]
    num = jnp.sum(zp[:, 0, :, :], axis=(0, 2))
    den = jnp.sum(zp[:, 1, :, :], axis=(0, 2))
    yhat = num / jnp.where(den > 0, den, 1.0) + b_bag_ref[0, 0]
    out_ref[...] = yhat.reshape(B, 1)


def _finalize(zp, b_bag):
    return pl.pallas_call(
        _fin_body,
        out_shape=jax.ShapeDtypeStruct((B, 1), jnp.float32),
    )(zp, b_bag.reshape(1, 1))


def kernel(x, bag_sizes, W_enc, b_enc, V, w_att, W_ins, b_ins, W_bag, b_bag):
    bs_pad = jnp.concatenate([bag_sizes[:B], bag_sizes[1:]])
    g = _encode(x, W_enc, b_enc, V, w_att, W_bag)
    zp = _sc_bag_sums(g, bs_pad)
    return _finalize(zp, b_bag)


# speedup vs baseline: 1.5563x; 1.0349x over previous
import functools

import jax
import jax.numpy as jnp
from jax import lax
from jax.experimental import pallas as pl
from jax.experimental.pallas import tpu as pltpu
from jax.experimental.pallas import tpu_sc as plsc

N = 32768
D_IN = 256
D_HID = 128
D_ATT = 64
B = 16
BLK = 2048
NBLK = N // BLK

_NC = 2
_NS = 16
_TILES = _NC * _NS
_RPT = N // _TILES
_LANE = 16
_VPT = _RPT // _LANE


def _enc_body(x_ref, W_enc_ref, b_enc_ref, V_ref, w_att_ref, W_bag_ref,
              g_ref):
    x = x_ref[...].astype(jnp.bfloat16)
    h = jnp.dot(x, W_enc_ref[...].astype(jnp.bfloat16),
                preferred_element_type=jnp.float32)
    h = h + b_enc_ref[...]
    t = jnp.tanh(jnp.dot(h.astype(jnp.bfloat16),
                         V_ref[...].astype(jnp.bfloat16),
                         preferred_element_type=jnp.float32))
    a_row = lax.dot_general(w_att_ref[...], t, (((0,), (1,)), ((), ())),
                            preferred_element_type=jnp.float32)
    e_row = jnp.exp(a_row)
    p_row = lax.dot_general(W_bag_ref[...], h, (((0,), (1,)), ((), ())),
                            preferred_element_type=jnp.float32)
    g_ref[...] = jnp.concatenate([e_row * p_row, e_row], axis=0)


def _encode(x, W_enc, b_enc, V, w_att, W_bag):
    return pl.pallas_call(
        _enc_body,
        grid=(NBLK,),
        in_specs=[
            pl.BlockSpec((BLK, D_IN), lambda i: (i, 0)),
            pl.BlockSpec((D_IN, D_HID), lambda i: (0, 0)),
            pl.BlockSpec((1, D_HID), lambda i: (0, 0)),
            pl.BlockSpec((D_HID, D_ATT), lambda i: (0, 0)),
            pl.BlockSpec((D_ATT, 1), lambda i: (0, 0)),
            pl.BlockSpec((D_HID, 1), lambda i: (0, 0)),
        ],
        out_specs=pl.BlockSpec((2, BLK), lambda i: (0, i)),
        out_shape=jax.ShapeDtypeStruct((2, N), jnp.float32),
    )(x, W_enc, b_enc.reshape(1, D_HID), V, w_att, W_bag)


_SC_MESH = plsc.VectorSubcoreMesh(core_axis_name="c", subcore_axis_name="s",
                                  num_cores=_NC, num_subcores=_NS)


@functools.partial(
    pl.kernel,
    out_type=jax.ShapeDtypeStruct((_TILES, 2, B, _LANE), jnp.float32),
    mesh=_SC_MESH,
    scratch_types=[
        pltpu.VMEM((_RPT,), jnp.float32),
        pltpu.VMEM((_RPT,), jnp.float32),
        pltpu.VMEM((B, _LANE), jnp.float32),
        pltpu.VMEM((B, _LANE), jnp.float32),
        pltpu.VMEM((2 * B,), jnp.int32),
    ],
)
def _sc_bag_sums(g_hbm, bs_hbm, out_hbm, wbuf, ebuf, acc_w, acc_e, bs_v):
    cid = lax.axis_index("c")
    sid = lax.axis_index("s")
    wid = sid * _NC + cid
    lo = wid * _RPT

    pltpu.sync_copy(bs_hbm, bs_v)
    pltpu.sync_copy(g_hbm.at[0, pl.ds(lo, _RPT)], wbuf)
    pltpu.sync_copy(g_hbm.at[1, pl.ds(lo, _RPT)], ebuf)
    bs_lo = bs_v[pl.ds(0, B)]
    bs_hi = bs_v[pl.ds(B, B)]

    lane = lax.iota(jnp.int32, _LANE)
    zero = jnp.zeros((_LANE,), jnp.float32)
    for j in range(B):
        acc_w[j] = zero
        acc_e[j] = zero

    for j in range(B):
        r0 = jnp.maximum(lo, bs_lo[j])
        r1 = jnp.minimum(lo + _RPT, bs_hi[j])

        @pl.when(r1 > r0)
        def _(j=j, r0=r0, r1=r1):
            v0 = (r0 - lo) // _LANE
            v1 = (r1 - lo + _LANE - 1) // _LANE

            def body(v, accs):
                aw, ae = accs
                base = v * _LANE
                idx = lo + base + lane
                m = (idx >= r0) & (idx < r1)
                wv = wbuf[pl.ds(base, _LANE)]
                ev = ebuf[pl.ds(base, _LANE)]
                return (aw + jnp.where(m, wv, 0.0),
                        ae + jnp.where(m, ev, 0.0))

            aw, ae = lax.fori_loop(v0, v1, body, (zero, zero))
            acc_w[j] = aw
            acc_e[j] = ae

    pltpu.sync_copy(acc_w, out_hbm.at[wid, 0])
    pltpu.sync_copy(acc_e, out_hbm.at[wid, 1])


def _fin_body(zp_ref, b_bag_ref, out_ref):
    zp = zp_ref[...]
    num = jnp.sum(zp[:, 0, :, :], axis=(0, 2))
    den = jnp.sum(zp[:, 1, :, :], axis=(0, 2))
    yhat = num / jnp.where(den > 0, den, 1.0) + b_bag_ref[0, 0]
    out_ref[...] = yhat.reshape(B, 1)


def _finalize(zp, b_bag):
    return pl.pallas_call(
        _fin_body,
        out_shape=jax.ShapeDtypeStruct((B, 1), jnp.float32),
    )(zp, b_bag.reshape(1, 1))


def kernel(x, bag_sizes, W_enc, b_enc, V, w_att, W_ins, b_ins, W_bag, b_bag):
    bs_pad = jnp.concatenate([bag_sizes[:B], bag_sizes[1:]])
    g = _encode(x, W_enc, b_enc, V, w_att, W_bag)
    zp = _sc_bag_sums(g, bs_pad)
    return _finalize(zp, b_bag)
